# baseline (speedup 1.0000x reference)
import functools

import jax
import jax.numpy as jnp
from jax import lax
from jax.experimental import pallas as pl
from jax.experimental.pallas import tpu as pltpu
from jax.experimental.pallas import tpu_sc as plsc

N = 10000
NPAD = 10240
E = 320000
EPAD = 327680
EB = 128
PE = 160000
PEPAD = 163840
NC, NS = 2, 16
F32 = jnp.float32
I32 = jnp.int32

_SC_PARAMS = dict(
    mesh=plsc.VectorSubcoreMesh(core_axis_name="c", subcore_axis_name="s"),
    compiler_params=pltpu.CompilerParams(needs_layout_passes=False),
)


def _zero_fill(ref, n16):
    zeros = jnp.zeros((16,), F32)

    def body(i, _):
        ref[i // 8, pl.ds((i % 8) * 16, 16)] = zeros
        return 0

    lax.fori_loop(0, n16, body, 0)


def _deg_body(dst_hbm, iden_hbm, deg_hbm, idx_v, hist_v, iden_v, acc_sh):
    c = lax.axis_index("c")
    s = lax.axis_index("s")
    half = NPAD // NC
    lo = c * half
    nrows = EPAD // EB // NS
    pltpu.sync_copy(dst_hbm.at[pl.ds(s * nrows, nrows)], idx_v)
    _zero_fill(hist_v, half // 16)

    @pl.when(s == 0)
    def _():
        pltpu.sync_copy(hist_v, acc_sh)

    pltpu.sync_copy(iden_hbm, iden_v)
    ones = jnp.ones((16,), F32)

    def ebody(i, _):
        idx = idx_v[i // 8, pl.ds((i % 8) * 16, 16)]
        il = idx - lo
        m = (il >= 0) & (il < half)
        ilc = jnp.where(m, il, 0)
        r = lax.shift_right_logical(ilc, 7)
        col = lax.bitwise_and(ilc, 127)
        plsc.addupdate_scatter(hist_v, [r, col], ones, mask=m)
        return 0

    lax.fori_loop(0, nrows * 8, ebody, 0)
    plsc.subcore_barrier()
    pltpu.sync_copy(hist_v, acc_sh.at[iden_v], add=True)
    plsc.subcore_barrier()

    @pl.when(s == 0)
    def _():
        pltpu.sync_copy(acc_sh, deg_hbm.at[c])


@functools.partial(
    pl.kernel,
    out_type=jax.ShapeDtypeStruct((NC, NPAD // NC // 128, 128), F32),
    scratch_types=[
        pltpu.VMEM((EPAD // EB // NS, EB), I32),
        pltpu.VMEM((NPAD // NC // 128, 128), F32),
        pltpu.VMEM((NPAD // NC // 128,), I32),
        pltpu.VMEM_SHARED((NPAD // NC // 128, 128), F32),
    ],
    **_SC_PARAMS,
)
def _deg(dst_hbm, iden_hbm, deg_hbm, idx_v, hist_v, iden_v, acc_sh):
    _deg_body(dst_hbm, iden_hbm, deg_hbm, idx_v, hist_v, iden_v, acc_sh)


def _mm1_body(deg_ref, x_ref, w1_ref, hs_ref, dis_ref):
    dis = lax.rsqrt(deg_ref[...] + 1.0)
    h = jnp.dot(x_ref[...], w1_ref[...], preferred_element_type=F32)
    hs = h * dis
    hs_ref[0] = hs[:, :128]
    hs_ref[1] = hs[:, 128:]
    dis_ref[...] = dis


def _mm1(deg, xp, w1):
    R = 1024
    g = NPAD // R
    return pl.pallas_call(
        _mm1_body,
        grid=(g,),
        in_specs=[
            pl.BlockSpec((R, 1), lambda i: (i, 0)),
            pl.BlockSpec((R, 128), lambda i: (i, 0)),
            pl.BlockSpec((128, 256), lambda i: (0, 0)),
        ],
        out_specs=[
            pl.BlockSpec((2, R, 128), lambda i: (0, i, 0)),
            pl.BlockSpec((R, 1), lambda i: (i, 0)),
        ],
        out_shape=[
            jax.ShapeDtypeStruct((2, NPAD, 128), F32),
            jax.ShapeDtypeStruct((NPAD, 1), F32),
        ],
    )(deg, xp, w1)


def _agg_pipeline(src_hbm, dst_hbm, tab_hbm, src_v, dst_v, buf0, buf1,
                  acc_sh, g0, g1, s0, s1, base, nb, ch, off):
    def chunk(q, _):
        pltpu.sync_copy(src_hbm.at[pl.ds(base + q * ch, ch)], src_v)
        pltpu.sync_copy(dst_hbm.at[pl.ds(base + q * ch, ch)], dst_v)

        if off is not None:
            def obody(i, _):
                sl = (i // 8, pl.ds((i % 8) * 16, 16))
                src_v[sl] = src_v[sl] + off
                return 0

            lax.fori_loop(0, ch * 8, obody, 0)

        pltpu.async_copy(tab_hbm.at[src_v.at[0]], buf0, g0)
        pltpu.async_copy(tab_hbm.at[src_v.at[1]], buf1, g1)

        def body(j2, _):
            j = 2 * j2
            pltpu.make_async_copy(tab_hbm.at[src_v.at[j]], buf0, g0).wait()
            pltpu.async_copy(buf0, acc_sh.at[dst_v.at[j]], s0, add=True)
            pltpu.make_async_copy(tab_hbm.at[src_v.at[j + 1]], buf1, g1).wait()
            pltpu.async_copy(buf1, acc_sh.at[dst_v.at[j + 1]], s1, add=True)
            pltpu.make_async_copy(buf0, acc_sh.at[dst_v.at[j]], s0).wait()

            @pl.when(j + 2 < ch)
            def _():
                pltpu.async_copy(tab_hbm.at[src_v.at[j + 2]], buf0, g0)

            pltpu.make_async_copy(buf1, acc_sh.at[dst_v.at[j + 1]], s1).wait()

            @pl.when(j + 3 < ch)
            def _():
                pltpu.async_copy(tab_hbm.at[src_v.at[j + 3]], buf1, g1)

            return 0

        lax.fori_loop(0, ch // 2, body, 0)
        return 0

    lax.fori_loop(0, nb // ch, chunk, 0)


def _agg_cols_body(src_hbm, dst_hbm, tab_hbm, out_hbm,
                   src_v, dst_v, buf0, buf1, zbuf_v, acc_sh, g0, g1, s0, s1):
    c = lax.axis_index("c")
    s = lax.axis_index("s")
    nb = EPAD // EB // NS
    rows = NPAD // NS
    _zero_fill(zbuf_v, 256)

    def zcopy(k, _):
        pltpu.sync_copy(zbuf_v, acc_sh.at[pl.ds(s * rows + k * 32, 32)])
        return 0

    lax.fori_loop(0, rows // 32, zcopy, 0)
    plsc.subcore_barrier()
    _agg_pipeline(src_hbm, dst_hbm, tab_hbm, src_v, dst_v, buf0, buf1,
                  acc_sh, g0, g1, s0, s1, s * nb, nb, 32, c * NPAD)
    plsc.subcore_barrier()
    pltpu.sync_copy(acc_sh.at[pl.ds(s * rows, rows)],
                    out_hbm.at[c, pl.ds(s * rows, rows)])


@functools.partial(
    pl.kernel,
    out_type=jax.ShapeDtypeStruct((2, NPAD, 128), F32),
    scratch_types=[
        pltpu.VMEM((32, EB), I32),
        pltpu.VMEM((32, EB), I32),
        pltpu.VMEM((EB, 128), F32),
        pltpu.VMEM((EB, 128), F32),
        pltpu.VMEM((32, 128), F32),
        pltpu.VMEM_SHARED((NPAD, 128), F32),
        pltpu.SemaphoreType.DMA,
        pltpu.SemaphoreType.DMA,
        pltpu.SemaphoreType.DMA,
        pltpu.SemaphoreType.DMA,
    ],
    **_SC_PARAMS,
)
def _agg_cols(src_hbm, dst_hbm, tab_hbm, out_hbm,
              src_v, dst_v, buf0, buf1, zbuf_v, acc_sh, g0, g1, s0, s1):
    _agg_cols_body(src_hbm, dst_hbm, tab_hbm, out_hbm,
                   src_v, dst_v, buf0, buf1, zbuf_v, acc_sh, g0, g1, s0, s1)


def _mm2_body(dis_ref, alo_ref, ahi_ref, hlo_ref, hhi_ref, b1_ref, w2_ref,
              out_ref):
    dis = dis_ref[...]
    b1 = b1_ref[...]
    zlo = jnp.maximum((alo_ref[0] + hlo_ref[0]) * dis + b1[:, :128], 0.0)
    zhi = jnp.maximum((ahi_ref[0] + hhi_ref[0]) * dis + b1[:, 128:], 0.0)
    h2 = (jnp.dot(zlo, w2_ref[...][:128], preferred_element_type=F32)
          + jnp.dot(zhi, w2_ref[...][128:], preferred_element_type=F32))
    out_ref[...] = h2 * dis


def _mm2(dis, agg3, hs3, b1, w2):
    R = 1024
    g = NPAD // R
    return pl.pallas_call(
        _mm2_body,
        grid=(g,),
        in_specs=[
            pl.BlockSpec((R, 1), lambda i: (i, 0)),
            pl.BlockSpec((1, R, 128), lambda i: (0, i, 0)),
            pl.BlockSpec((1, R, 128), lambda i: (1, i, 0)),
            pl.BlockSpec((1, R, 128), lambda i: (0, i, 0)),
            pl.BlockSpec((1, R, 128), lambda i: (1, i, 0)),
            pl.BlockSpec((1, 256), lambda i: (0, 0)),
            pl.BlockSpec((256, 128), lambda i: (0, 0)),
        ],
        out_specs=pl.BlockSpec((R, 128), lambda i: (i, 0)),
        out_shape=jax.ShapeDtypeStruct((NPAD, 128), F32),
    )(dis, agg3, agg3, hs3, hs3, b1, w2)


def _agg_rows_body(src_hbm, dst_hbm, tab_hbm, out_hbm,
                   src_v, dst_v, buf0, buf1, zbuf_v, acc_sh, g0, g1, s0, s1):
    c = lax.axis_index("c")
    s = lax.axis_index("s")
    nb = EPAD // EB // (NC * NS)
    rows = NPAD // NS
    _zero_fill(zbuf_v, 256)

    def zcopy(k, _):
        pltpu.sync_copy(zbuf_v, acc_sh.at[pl.ds(s * rows + k * 32, 32)])
        return 0

    lax.fori_loop(0, rows // 32, zcopy, 0)
    plsc.subcore_barrier()
    _agg_pipeline(src_hbm, dst_hbm, tab_hbm, src_v, dst_v, buf0, buf1,
                  acc_sh, g0, g1, s0, s1, (c * NS + s) * nb, nb, 16, None)
    plsc.subcore_barrier()
    pltpu.sync_copy(acc_sh.at[pl.ds(s * rows, rows)],
                    out_hbm.at[c, pl.ds(s * rows, rows)])


@functools.partial(
    pl.kernel,
    out_type=jax.ShapeDtypeStruct((2, NPAD, 128), F32),
    scratch_types=[
        pltpu.VMEM((16, EB), I32),
        pltpu.VMEM((16, EB), I32),
        pltpu.VMEM((EB, 128), F32),
        pltpu.VMEM((EB, 128), F32),
        pltpu.VMEM((32, 128), F32),
        pltpu.VMEM_SHARED((NPAD, 128), F32),
        pltpu.SemaphoreType.DMA,
        pltpu.SemaphoreType.DMA,
        pltpu.SemaphoreType.DMA,
        pltpu.SemaphoreType.DMA,
    ],
    **_SC_PARAMS,
)
def _agg_rows(src_hbm, dst_hbm, tab_hbm, out_hbm,
              src_v, dst_v, buf0, buf1, zbuf_v, acc_sh, g0, g1, s0, s1):
    _agg_rows_body(src_hbm, dst_hbm, tab_hbm, out_hbm,
                   src_v, dst_v, buf0, buf1, zbuf_v, acc_sh, g0, g1, s0, s1)


def _mm3_body(dis_ref, p0_ref, p1_ref, h2_ref, b2_ref, wt_ref, wb_ref, bl1_ref,
              z_ref, a_ref, bb_ref):
    dis = dis_ref[...]
    z = (p0_ref[0] + p1_ref[0] + h2_ref[...]) * dis + b2_ref[...]
    z_ref[...] = z
    a_ref[...] = jnp.dot(
        z, wt_ref[...], preferred_element_type=F32).astype(jnp.bfloat16)
    bb_ref[...] = (jnp.dot(z, wb_ref[...], preferred_element_type=F32)
                   + bl1_ref[...]).astype(jnp.bfloat16)


def _mm3(dis, p3, h2, b2, wt, wb, bl1):
    R = 1024
    g = NPAD // R
    return pl.pallas_call(
        _mm3_body,
        grid=(g,),
        in_specs=[
            pl.BlockSpec((R, 1), lambda i: (i, 0)),
            pl.BlockSpec((1, R, 128), lambda i: (0, i, 0)),
            pl.BlockSpec((1, R, 128), lambda i: (1, i, 0)),
            pl.BlockSpec((R, 128), lambda i: (i, 0)),
            pl.BlockSpec((1, 128), lambda i: (0, 0)),
            pl.BlockSpec((128, 256), lambda i: (0, 0)),
            pl.BlockSpec((128, 256), lambda i: (0, 0)),
            pl.BlockSpec((1, 256), lambda i: (0, 0)),
        ],
        out_specs=[
            pl.BlockSpec((R, 128), lambda i: (i, 0)),
            pl.BlockSpec((R, 256), lambda i: (i, 0)),
            pl.BlockSpec((R, 256), lambda i: (i, 0)),
        ],
        out_shape=[
            jax.ShapeDtypeStruct((NPAD, 128), F32),
            jax.ShapeDtypeStruct((NPAD, 256), jnp.bfloat16),
            jax.ShapeDtypeStruct((NPAD, 256), jnp.bfloat16),
        ],
    )(dis, p3, p3, h2, b2, wt, wb, bl1)


DB = 128
DNB = 2 * PEPAD // DB // (NC * NS)
BF16 = jnp.bfloat16


def _decode_body(sv_hbm, dv_hbm, a_hbm, b_hbm, wb_hbm, bl2_hbm,
                 out_hbm, sv, dv, bufa0, bufb0, bufa1, bufb1, wbv, bl2v_ref,
                 score_v, sa0, sb0, sa1, sb1):
    c = lax.axis_index("c")
    s = lax.axis_index("s")
    wid = c * NS + s
    pltpu.sync_copy(wb_hbm, wbv)
    pltpu.sync_copy(bl2_hbm, bl2v_ref)
    pltpu.sync_copy(sv_hbm.at[pl.ds(wid * DNB, DNB)], sv)
    pltpu.sync_copy(dv_hbm.at[pl.ds(wid * DNB, DNB)], dv)
    bl2v = bl2v_ref[0]
    lane = lax.iota(I32, 16)
    zero = jnp.zeros((16,), F32)
    wev = [wbv[2 * wc] for wc in range(8)]
    wod = [wbv[2 * wc + 1] for wc in range(8)]

    def compute(j, bufa, bufb):
        def ebody(e, svec):
            acc_e = zero
            acc_o = zero
            for wc in range(8):
                wa = bufa[e, pl.ds(wc * 16, 16)]
                wb = bufb[e, pl.ds(wc * 16, 16)]
                u = jnp.maximum(plsc.bitcast(wa, BF16)
                                + plsc.bitcast(wb, BF16), 0.0)
                m = plsc.bitcast(u, I32)
                ue = plsc.bitcast(lax.shift_left(m, 16), F32)
                uo = plsc.bitcast(m & jnp.int32(-65536), F32)
                acc_e = acc_e + ue * wev[wc]
                acc_o = acc_o + uo * wod[wc]
            sc = jnp.sum(acc_e + acc_o)
            svec = jnp.where(lane == (e & 15), sc, svec)

            @pl.when((e & 15) == 15)
            def _():
                gidx = lax.shift_left(lax.shift_right_logical(e, 4), 4)
                score_v[j & 15, pl.ds(gidx, 16)] = (
                    1.0 / (1.0 + jnp.exp(-(svec + bl2v))))

            return svec

        lax.fori_loop(0, DB, ebody, zero)

    pltpu.async_copy(a_hbm.at[sv.at[0]], bufa0, sa0)
    pltpu.async_copy(b_hbm.at[dv.at[0]], bufb0, sb0)

    def body(j2, _):
        j = 2 * j2
        pltpu.async_copy(a_hbm.at[sv.at[j + 1]], bufa1, sa1)
        pltpu.async_copy(b_hbm.at[dv.at[j + 1]], bufb1, sb1)
        pltpu.make_async_copy(a_hbm.at[sv.at[j]], bufa0, sa0).wait()
        pltpu.make_async_copy(b_hbm.at[dv.at[j]], bufb0, sb0).wait()
        compute(j, bufa0, bufb0)

        @pl.when(j + 2 < DNB)
        def _():
            pltpu.async_copy(a_hbm.at[sv.at[j + 2]], bufa0, sa0)
            pltpu.async_copy(b_hbm.at[dv.at[j + 2]], bufb0, sb0)

        pltpu.make_async_copy(a_hbm.at[sv.at[j + 1]], bufa1, sa1).wait()
        pltpu.make_async_copy(b_hbm.at[dv.at[j + 1]], bufb1, sb1).wait()
        compute(j + 1, bufa1, bufb1)

        @pl.when((j2 & 7) == 7)
        def _():
            start = pl.multiple_of(wid * DNB + 2 * j2 - 14, 8)
            pltpu.sync_copy(score_v, out_hbm.at[pl.ds(start, 16)])

        return 0

    lax.fori_loop(0, DNB // 2, body, 0)


@functools.partial(
    pl.kernel,
    out_type=jax.ShapeDtypeStruct((2 * PEPAD // DB, DB), F32),
    scratch_types=[
        pltpu.VMEM((DNB, DB), I32),
        pltpu.VMEM((DNB, DB), I32),
        pltpu.VMEM((DB, 128), I32),
        pltpu.VMEM((DB, 128), I32),
        pltpu.VMEM((DB, 128), I32),
        pltpu.VMEM((DB, 128), I32),
        pltpu.VMEM((16, 16), F32),
        pltpu.VMEM((1, 16), F32),
        pltpu.VMEM((16, DB), F32),
        pltpu.SemaphoreType.DMA,
        pltpu.SemaphoreType.DMA,
        pltpu.SemaphoreType.DMA,
        pltpu.SemaphoreType.DMA,
    ],
    **_SC_PARAMS,
)
def _decode(sv_hbm, dv_hbm, a_hbm, b_hbm, wb_hbm, bl2_hbm,
            out_hbm, sv, dv, bufa0, bufb0, bufa1, bufb1, wbv, bl2v_ref,
            score_v, sa0, sb0, sa1, sb1):
    _decode_body(sv_hbm, dv_hbm, a_hbm, b_hbm, wb_hbm, bl2_hbm,
                 out_hbm, sv, dv, bufa0, bufb0, bufa1, bufb1, wbv, bl2v_ref,
                 score_v, sa0, sb0, sa1, sb1)


def kernel(x, edge_index, pos_edge_index, neg_edge_index,
           W1, b1, W2, b2, Wl1, bl1, Wl2, bl2):
    epad = jnp.full((EPAD - E,), N, I32)
    src2 = jnp.concatenate([edge_index[0], epad]).reshape(EPAD // EB, EB)
    dst2 = jnp.concatenate([edge_index[1], epad]).reshape(EPAD // EB, EB)
    xp = jnp.pad(x, ((0, NPAD - N), (0, 0)))

    iden = jnp.arange(NPAD // NC // 128, dtype=I32)
    deg = _deg(dst2, iden).reshape(NPAD, 1)
    hs3, dis = _mm1(deg, xp, W1)
    agg3 = _agg_cols(src2, dst2, hs3.reshape(2 * NPAD, 128))
    h2 = _mm2(dis, agg3, hs3, b1.reshape(1, 256), W2)
    p3 = _agg_rows(src2, dst2, h2)
    z, A, Bb = _mm3(dis, p3, h2, b2.reshape(1, 128), Wl1[:128], Wl1[128:],
                    bl1.reshape(1, 256))

    ppad = jnp.zeros((PEPAD - PE,), I32)
    sv2 = jnp.concatenate([pos_edge_index[0], ppad, neg_edge_index[0], ppad]
                          ).reshape(2 * PEPAD // DB, DB)
    dv2 = jnp.concatenate([pos_edge_index[1], ppad, neg_edge_index[1], ppad]
                          ).reshape(2 * PEPAD // DB, DB)
    apk = lax.bitcast_convert_type(A.reshape(NPAD, 128, 2), I32)
    bpk = lax.bitcast_convert_type(Bb.reshape(NPAD, 128, 2), I32)
    wbc = Wl2[:, 0].reshape(8, 16, 2).transpose(0, 2, 1).reshape(16, 16)
    bl2v = jnp.broadcast_to(bl2, (1, 16))

    scores = _decode(sv2, dv2, apk, bpk, wbc, bl2v).reshape(-1)
    return (z[:N], scores[:PE], scores[PEPAD:PEPAD + PE])


# speedup vs baseline: 1.0512x; 1.0512x over previous
import functools

import jax
import jax.numpy as jnp
from jax import lax
from jax.experimental import pallas as pl
from jax.experimental.pallas import tpu as pltpu
from jax.experimental.pallas import tpu_sc as plsc

N = 10000
NPAD = 10240
E = 320000
EPAD = 327680
EB = 128
PE = 160000
PEPAD = 163840
NC, NS = 2, 16
F32 = jnp.float32
I32 = jnp.int32

_SC_PARAMS = dict(
    mesh=plsc.VectorSubcoreMesh(core_axis_name="c", subcore_axis_name="s"),
    compiler_params=pltpu.CompilerParams(needs_layout_passes=False),
)


def _zero_fill(ref, n16):
    zeros = jnp.zeros((16,), F32)

    def body(i, _):
        ref[i // 8, pl.ds((i % 8) * 16, 16)] = zeros
        return 0

    lax.fori_loop(0, n16, body, 0)


def _deg_body(dst_hbm, iden_hbm, deg_hbm, idx_v, hist_v, iden_v, acc_sh):
    c = lax.axis_index("c")
    s = lax.axis_index("s")
    half = NPAD // NC
    lo = c * half
    nrows = EPAD // EB // NS
    pltpu.sync_copy(dst_hbm.at[pl.ds(s * nrows, nrows)], idx_v)
    _zero_fill(hist_v, half // 16)

    @pl.when(s == 0)
    def _():
        pltpu.sync_copy(hist_v, acc_sh)

    pltpu.sync_copy(iden_hbm, iden_v)
    ones = jnp.ones((16,), F32)

    def ebody(i, _):
        idx = idx_v[i // 8, pl.ds((i % 8) * 16, 16)]
        il = idx - lo
        m = (il >= 0) & (il < half)
        ilc = jnp.where(m, il, 0)
        r = lax.shift_right_logical(ilc, 7)
        col = lax.bitwise_and(ilc, 127)
        plsc.addupdate_scatter(hist_v, [r, col], ones, mask=m)
        return 0

    lax.fori_loop(0, nrows * 8, ebody, 0)
    plsc.subcore_barrier()
    pltpu.sync_copy(hist_v, acc_sh.at[iden_v], add=True)
    plsc.subcore_barrier()

    @pl.when(s == 0)
    def _():
        pltpu.sync_copy(acc_sh, deg_hbm.at[c])


@functools.partial(
    pl.kernel,
    out_type=jax.ShapeDtypeStruct((NC, NPAD // NC // 128, 128), F32),
    scratch_types=[
        pltpu.VMEM((EPAD // EB // NS, EB), I32),
        pltpu.VMEM((NPAD // NC // 128, 128), F32),
        pltpu.VMEM((NPAD // NC // 128,), I32),
        pltpu.VMEM_SHARED((NPAD // NC // 128, 128), F32),
    ],
    **_SC_PARAMS,
)
def _deg(dst_hbm, iden_hbm, deg_hbm, idx_v, hist_v, iden_v, acc_sh):
    _deg_body(dst_hbm, iden_hbm, deg_hbm, idx_v, hist_v, iden_v, acc_sh)


def _mm1_body(deg_ref, x_ref, w1_ref, hs_ref, dis_ref):
    dis = lax.rsqrt(deg_ref[...] + 1.0)
    h = jnp.dot(x_ref[...], w1_ref[...], preferred_element_type=F32)
    hs = h * dis
    hs_ref[0] = hs[:, :128]
    hs_ref[1] = hs[:, 128:]
    dis_ref[...] = dis


def _mm1(deg, xp, w1):
    R = 1024
    g = NPAD // R
    return pl.pallas_call(
        _mm1_body,
        grid=(g,),
        in_specs=[
            pl.BlockSpec((R, 1), lambda i: (i, 0)),
            pl.BlockSpec((R, 128), lambda i: (i, 0)),
            pl.BlockSpec((128, 256), lambda i: (0, 0)),
        ],
        out_specs=[
            pl.BlockSpec((2, R, 128), lambda i: (0, i, 0)),
            pl.BlockSpec((R, 1), lambda i: (i, 0)),
        ],
        out_shape=[
            jax.ShapeDtypeStruct((2, NPAD, 128), F32),
            jax.ShapeDtypeStruct((NPAD, 1), F32),
        ],
    )(deg, xp, w1)


def _agg_pipeline(src_hbm, dst_hbm, tab_hbm, src_v, dst_v, buf0, buf1,
                  acc_sh, g0, g1, s0, s1, base, nb, ch, off):
    def chunk(q, _):
        pltpu.sync_copy(src_hbm.at[pl.ds(base + q * ch, ch)], src_v)
        pltpu.sync_copy(dst_hbm.at[pl.ds(base + q * ch, ch)], dst_v)

        if off is not None:
            def obody(i, _):
                sl = (i // 8, pl.ds((i % 8) * 16, 16))
                src_v[sl] = src_v[sl] + off
                return 0

            lax.fori_loop(0, ch * 8, obody, 0)

        pltpu.async_copy(tab_hbm.at[src_v.at[0]], buf0, g0)
        pltpu.async_copy(tab_hbm.at[src_v.at[1]], buf1, g1)

        def body(j2, _):
            j = 2 * j2
            pltpu.make_async_copy(tab_hbm.at[src_v.at[j]], buf0, g0).wait()
            pltpu.async_copy(buf0, acc_sh.at[dst_v.at[j]], s0, add=True)
            pltpu.make_async_copy(tab_hbm.at[src_v.at[j + 1]], buf1, g1).wait()
            pltpu.async_copy(buf1, acc_sh.at[dst_v.at[j + 1]], s1, add=True)
            pltpu.make_async_copy(buf0, acc_sh.at[dst_v.at[j]], s0).wait()

            @pl.when(j + 2 < ch)
            def _():
                pltpu.async_copy(tab_hbm.at[src_v.at[j + 2]], buf0, g0)

            pltpu.make_async_copy(buf1, acc_sh.at[dst_v.at[j + 1]], s1).wait()

            @pl.when(j + 3 < ch)
            def _():
                pltpu.async_copy(tab_hbm.at[src_v.at[j + 3]], buf1, g1)

            return 0

        lax.fori_loop(0, ch // 2, body, 0)
        return 0

    lax.fori_loop(0, nb // ch, chunk, 0)


def _agg_cols_body(src_hbm, dst_hbm, tab_hbm, out_hbm,
                   src_v, dst_v, buf0, buf1, zbuf_v, acc_sh, g0, g1, s0, s1):
    c = lax.axis_index("c")
    s = lax.axis_index("s")
    nb = EPAD // EB // NS
    rows = NPAD // NS
    _zero_fill(zbuf_v, 256)

    def zcopy(k, _):
        pltpu.sync_copy(zbuf_v, acc_sh.at[pl.ds(s * rows + k * 32, 32)])
        return 0

    lax.fori_loop(0, rows // 32, zcopy, 0)
    plsc.subcore_barrier()
    _agg_pipeline(src_hbm, dst_hbm, tab_hbm, src_v, dst_v, buf0, buf1,
                  acc_sh, g0, g1, s0, s1, s * nb, nb, 32, c * NPAD)
    plsc.subcore_barrier()
    pltpu.sync_copy(acc_sh.at[pl.ds(s * rows, rows)],
                    out_hbm.at[c, pl.ds(s * rows, rows)])


@functools.partial(
    pl.kernel,
    out_type=jax.ShapeDtypeStruct((2, NPAD, 128), F32),
    scratch_types=[
        pltpu.VMEM((32, EB), I32),
        pltpu.VMEM((32, EB), I32),
        pltpu.VMEM((EB, 128), F32),
        pltpu.VMEM((EB, 128), F32),
        pltpu.VMEM((32, 128), F32),
        pltpu.VMEM_SHARED((NPAD, 128), F32),
        pltpu.SemaphoreType.DMA,
        pltpu.SemaphoreType.DMA,
        pltpu.SemaphoreType.DMA,
        pltpu.SemaphoreType.DMA,
    ],
    **_SC_PARAMS,
)
def _agg_cols(src_hbm, dst_hbm, tab_hbm, out_hbm,
              src_v, dst_v, buf0, buf1, zbuf_v, acc_sh, g0, g1, s0, s1):
    _agg_cols_body(src_hbm, dst_hbm, tab_hbm, out_hbm,
                   src_v, dst_v, buf0, buf1, zbuf_v, acc_sh, g0, g1, s0, s1)


def _mm2_body(dis_ref, alo_ref, ahi_ref, hlo_ref, hhi_ref, b1_ref, w2_ref,
              out_ref):
    dis = dis_ref[...]
    b1 = b1_ref[...]
    zlo = jnp.maximum((alo_ref[0] + hlo_ref[0]) * dis + b1[:, :128], 0.0)
    zhi = jnp.maximum((ahi_ref[0] + hhi_ref[0]) * dis + b1[:, 128:], 0.0)
    h2 = (jnp.dot(zlo, w2_ref[...][:128], preferred_element_type=F32)
          + jnp.dot(zhi, w2_ref[...][128:], preferred_element_type=F32))
    out_ref[...] = h2 * dis


def _mm2(dis, agg3, hs3, b1, w2):
    R = 1024
    g = NPAD // R
    return pl.pallas_call(
        _mm2_body,
        grid=(g,),
        in_specs=[
            pl.BlockSpec((R, 1), lambda i: (i, 0)),
            pl.BlockSpec((1, R, 128), lambda i: (0, i, 0)),
            pl.BlockSpec((1, R, 128), lambda i: (1, i, 0)),
            pl.BlockSpec((1, R, 128), lambda i: (0, i, 0)),
            pl.BlockSpec((1, R, 128), lambda i: (1, i, 0)),
            pl.BlockSpec((1, 256), lambda i: (0, 0)),
            pl.BlockSpec((256, 128), lambda i: (0, 0)),
        ],
        out_specs=pl.BlockSpec((R, 128), lambda i: (i, 0)),
        out_shape=jax.ShapeDtypeStruct((NPAD, 128), F32),
    )(dis, agg3, agg3, hs3, hs3, b1, w2)


def _agg_rows_body(src_hbm, dst_hbm, tab_hbm, out_hbm,
                   src_v, dst_v, buf0, buf1, zbuf_v, acc_sh, g0, g1, s0, s1):
    c = lax.axis_index("c")
    s = lax.axis_index("s")
    nb = EPAD // EB // (NC * NS)
    rows = NPAD // NS
    _zero_fill(zbuf_v, 256)

    def zcopy(k, _):
        pltpu.sync_copy(zbuf_v, acc_sh.at[pl.ds(s * rows + k * 32, 32)])
        return 0

    lax.fori_loop(0, rows // 32, zcopy, 0)
    plsc.subcore_barrier()
    _agg_pipeline(src_hbm, dst_hbm, tab_hbm, src_v, dst_v, buf0, buf1,
                  acc_sh, g0, g1, s0, s1, (c * NS + s) * nb, nb, 16, None)
    plsc.subcore_barrier()
    pltpu.sync_copy(acc_sh.at[pl.ds(s * rows, rows)],
                    out_hbm.at[c, pl.ds(s * rows, rows)])


@functools.partial(
    pl.kernel,
    out_type=jax.ShapeDtypeStruct((2, NPAD, 128), F32),
    scratch_types=[
        pltpu.VMEM((16, EB), I32),
        pltpu.VMEM((16, EB), I32),
        pltpu.VMEM((EB, 128), F32),
        pltpu.VMEM((EB, 128), F32),
        pltpu.VMEM((32, 128), F32),
        pltpu.VMEM_SHARED((NPAD, 128), F32),
        pltpu.SemaphoreType.DMA,
        pltpu.SemaphoreType.DMA,
        pltpu.SemaphoreType.DMA,
        pltpu.SemaphoreType.DMA,
    ],
    **_SC_PARAMS,
)
def _agg_rows(src_hbm, dst_hbm, tab_hbm, out_hbm,
              src_v, dst_v, buf0, buf1, zbuf_v, acc_sh, g0, g1, s0, s1):
    _agg_rows_body(src_hbm, dst_hbm, tab_hbm, out_hbm,
                   src_v, dst_v, buf0, buf1, zbuf_v, acc_sh, g0, g1, s0, s1)


def _mm3_body(dis_ref, p0_ref, p1_ref, h2_ref, b2_ref, wt_ref, wb_ref, bl1_ref,
              z_ref, a_ref, bb_ref):
    dis = dis_ref[...]
    z = (p0_ref[0] + p1_ref[0] + h2_ref[...]) * dis + b2_ref[...]
    z_ref[...] = z
    a_ref[...] = jnp.dot(
        z, wt_ref[...], preferred_element_type=F32).astype(jnp.bfloat16)
    bb_ref[...] = (jnp.dot(z, wb_ref[...], preferred_element_type=F32)
                   + bl1_ref[...]).astype(jnp.bfloat16)


def _mm3(dis, p3, h2, b2, wt, wb, bl1):
    R = 1024
    g = NPAD // R
    return pl.pallas_call(
        _mm3_body,
        grid=(g,),
        in_specs=[
            pl.BlockSpec((R, 1), lambda i: (i, 0)),
            pl.BlockSpec((1, R, 128), lambda i: (0, i, 0)),
            pl.BlockSpec((1, R, 128), lambda i: (1, i, 0)),
            pl.BlockSpec((R, 128), lambda i: (i, 0)),
            pl.BlockSpec((1, 128), lambda i: (0, 0)),
            pl.BlockSpec((128, 256), lambda i: (0, 0)),
            pl.BlockSpec((128, 256), lambda i: (0, 0)),
            pl.BlockSpec((1, 256), lambda i: (0, 0)),
        ],
        out_specs=[
            pl.BlockSpec((R, 128), lambda i: (i, 0)),
            pl.BlockSpec((R, 256), lambda i: (i, 0)),
            pl.BlockSpec((R, 256), lambda i: (i, 0)),
        ],
        out_shape=[
            jax.ShapeDtypeStruct((NPAD, 128), F32),
            jax.ShapeDtypeStruct((NPAD, 256), jnp.bfloat16),
            jax.ShapeDtypeStruct((NPAD, 256), jnp.bfloat16),
        ],
    )(dis, p3, p3, h2, b2, wt, wb, bl1)


DB = 128
DNB = 2 * PEPAD // DB // (NC * NS)
BF16 = jnp.bfloat16


def _decode_body(sv_hbm, dv_hbm, a_hbm, b_hbm, wb_hbm, bl2_hbm,
                 out_hbm, sv, dv, bufa0, bufb0, bufa1, bufb1, wbv, bl2v_ref,
                 score_v, sa0, sb0, sa1, sb1):
    c = lax.axis_index("c")
    s = lax.axis_index("s")
    wid = c * NS + s
    pltpu.sync_copy(wb_hbm, wbv)
    pltpu.sync_copy(bl2_hbm, bl2v_ref)
    pltpu.sync_copy(sv_hbm.at[pl.ds(wid * DNB, DNB)], sv)
    pltpu.sync_copy(dv_hbm.at[pl.ds(wid * DNB, DNB)], dv)
    bl2v = bl2v_ref[0]
    lane = lax.iota(I32, 16)
    zero = jnp.zeros((16,), F32)
    wev = [wbv[2 * wc] for wc in range(8)]
    wod = [wbv[2 * wc + 1] for wc in range(8)]

    def compute(j, bufa, bufb):
        def dot1(e):
            acc_e = zero
            acc_o = zero
            for wc in range(8):
                wa = bufa[e, pl.ds(wc * 16, 16)]
                wb = bufb[e, pl.ds(wc * 16, 16)]
                u = jnp.maximum(plsc.bitcast(wa, BF16)
                                + plsc.bitcast(wb, BF16), 0.0)
                m = plsc.bitcast(u, I32)
                ue = plsc.bitcast(lax.shift_left(m, 16), F32)
                uo = plsc.bitcast(m & jnp.int32(-65536), F32)
                acc_e = acc_e + ue * wev[wc]
                acc_o = acc_o + uo * wod[wc]
            return jnp.sum(acc_e + acc_o)

        def ebody(e2, svec):
            e = 2 * e2
            s0 = dot1(e)
            s1 = dot1(e + 1)
            svec = jnp.where(lane == (e & 15), s0, svec)
            svec = jnp.where(lane == ((e + 1) & 15), s1, svec)

            @pl.when(((e + 1) & 15) == 15)
            def _():
                gidx = lax.shift_left(lax.shift_right_logical(e, 4), 4)
                score_v[j & 15, pl.ds(gidx, 16)] = (
                    1.0 / (1.0 + jnp.exp(-(svec + bl2v))))

            return svec

        lax.fori_loop(0, DB // 2, ebody, zero)

    pltpu.async_copy(a_hbm.at[sv.at[0]], bufa0, sa0)
    pltpu.async_copy(b_hbm.at[dv.at[0]], bufb0, sb0)

    def body(j2, _):
        j = 2 * j2
        pltpu.async_copy(a_hbm.at[sv.at[j + 1]], bufa1, sa1)
        pltpu.async_copy(b_hbm.at[dv.at[j + 1]], bufb1, sb1)
        pltpu.make_async_copy(a_hbm.at[sv.at[j]], bufa0, sa0).wait()
        pltpu.make_async_copy(b_hbm.at[dv.at[j]], bufb0, sb0).wait()
        compute(j, bufa0, bufb0)

        @pl.when(j + 2 < DNB)
        def _():
            pltpu.async_copy(a_hbm.at[sv.at[j + 2]], bufa0, sa0)
            pltpu.async_copy(b_hbm.at[dv.at[j + 2]], bufb0, sb0)

        pltpu.make_async_copy(a_hbm.at[sv.at[j + 1]], bufa1, sa1).wait()
        pltpu.make_async_copy(b_hbm.at[dv.at[j + 1]], bufb1, sb1).wait()
        compute(j + 1, bufa1, bufb1)

        @pl.when((j2 & 7) == 7)
        def _():
            start = pl.multiple_of(wid * DNB + 2 * j2 - 14, 8)
            pltpu.sync_copy(score_v, out_hbm.at[pl.ds(start, 16)])

        return 0

    lax.fori_loop(0, DNB // 2, body, 0)


@functools.partial(
    pl.kernel,
    out_type=jax.ShapeDtypeStruct((2 * PEPAD // DB, DB), F32),
    scratch_types=[
        pltpu.VMEM((DNB, DB), I32),
        pltpu.VMEM((DNB, DB), I32),
        pltpu.VMEM((DB, 128), I32),
        pltpu.VMEM((DB, 128), I32),
        pltpu.VMEM((DB, 128), I32),
        pltpu.VMEM((DB, 128), I32),
        pltpu.VMEM((16, 16), F32),
        pltpu.VMEM((1, 16), F32),
        pltpu.VMEM((16, DB), F32),
        pltpu.SemaphoreType.DMA,
        pltpu.SemaphoreType.DMA,
        pltpu.SemaphoreType.DMA,
        pltpu.SemaphoreType.DMA,
    ],
    **_SC_PARAMS,
)
def _decode(sv_hbm, dv_hbm, a_hbm, b_hbm, wb_hbm, bl2_hbm,
            out_hbm, sv, dv, bufa0, bufb0, bufa1, bufb1, wbv, bl2v_ref,
            score_v, sa0, sb0, sa1, sb1):
    _decode_body(sv_hbm, dv_hbm, a_hbm, b_hbm, wb_hbm, bl2_hbm,
                 out_hbm, sv, dv, bufa0, bufb0, bufa1, bufb1, wbv, bl2v_ref,
                 score_v, sa0, sb0, sa1, sb1)


def kernel(x, edge_index, pos_edge_index, neg_edge_index,
           W1, b1, W2, b2, Wl1, bl1, Wl2, bl2):
    epad = jnp.full((EPAD - E,), N, I32)
    src2 = jnp.concatenate([edge_index[0], epad]).reshape(EPAD // EB, EB)
    dst2 = jnp.concatenate([edge_index[1], epad]).reshape(EPAD // EB, EB)
    xp = jnp.pad(x, ((0, NPAD - N), (0, 0)))

    iden = jnp.arange(NPAD // NC // 128, dtype=I32)
    deg = _deg(dst2, iden).reshape(NPAD, 1)
    hs3, dis = _mm1(deg, xp, W1)
    agg3 = _agg_cols(src2, dst2, hs3.reshape(2 * NPAD, 128))
    h2 = _mm2(dis, agg3, hs3, b1.reshape(1, 256), W2)
    p3 = _agg_rows(src2, dst2, h2)
    z, A, Bb = _mm3(dis, p3, h2, b2.reshape(1, 128), Wl1[:128], Wl1[128:],
                    bl1.reshape(1, 256))

    ppad = jnp.zeros((PEPAD - PE,), I32)
    sv2 = jnp.concatenate([pos_edge_index[0], ppad, neg_edge_index[0], ppad]
                          ).reshape(2 * PEPAD // DB, DB)
    dv2 = jnp.concatenate([pos_edge_index[1], ppad, neg_edge_index[1], ppad]
                          ).reshape(2 * PEPAD // DB, DB)
    apk = lax.bitcast_convert_type(A.reshape(NPAD, 128, 2), I32)
    bpk = lax.bitcast_convert_type(Bb.reshape(NPAD, 128, 2), I32)
    wbc = Wl2[:, 0].reshape(8, 16, 2).transpose(0, 2, 1).reshape(16, 16)
    bl2v = jnp.broadcast_to(bl2, (1, 16))

    scores = _decode(sv2, dv2, apk, bpk, wbc, bl2v).reshape(-1)
    return (z[:N], scores[:PE], scores[PEPAD:PEPAD + PE])


# speedup vs baseline: 1.0735x; 1.0212x over previous
import functools

import jax
import jax.numpy as jnp
from jax import lax
from jax.experimental import pallas as pl
from jax.experimental.pallas import tpu as pltpu
from jax.experimental.pallas import tpu_sc as plsc

N = 10000
NPAD = 10240
E = 320000
EPAD = 327680
EB = 128
PE = 160000
PEPAD = 163840
NC, NS = 2, 16
F32 = jnp.float32
I32 = jnp.int32

_SC_PARAMS = dict(
    mesh=plsc.VectorSubcoreMesh(core_axis_name="c", subcore_axis_name="s"),
    compiler_params=pltpu.CompilerParams(needs_layout_passes=False),
)


def _zero_fill(ref, n16):
    zeros = jnp.zeros((16,), F32)

    def body(i, _):
        ref[i // 8, pl.ds((i % 8) * 16, 16)] = zeros
        return 0

    lax.fori_loop(0, n16, body, 0)


def _deg_body(dst_hbm, iden_hbm, deg_hbm, idx_v, hist_v, iden_v, acc_sh):
    c = lax.axis_index("c")
    s = lax.axis_index("s")
    half = NPAD // NC
    lo = c * half
    nrows = EPAD // EB // NS
    pltpu.sync_copy(dst_hbm.at[pl.ds(s * nrows, nrows)], idx_v)
    _zero_fill(hist_v, half // 16)

    @pl.when(s == 0)
    def _():
        pltpu.sync_copy(hist_v, acc_sh)

    pltpu.sync_copy(iden_hbm, iden_v)
    ones = jnp.ones((16,), F32)

    def ebody(i, _):
        idx = idx_v[i // 8, pl.ds((i % 8) * 16, 16)]
        il = idx - lo
        m = (il >= 0) & (il < half)
        ilc = jnp.where(m, il, 0)
        r = lax.shift_right_logical(ilc, 7)
        col = lax.bitwise_and(ilc, 127)
        plsc.addupdate_scatter(hist_v, [r, col], ones, mask=m)
        return 0

    lax.fori_loop(0, nrows * 8, ebody, 0)
    plsc.subcore_barrier()
    pltpu.sync_copy(hist_v, acc_sh.at[iden_v], add=True)
    plsc.subcore_barrier()

    @pl.when(s == 0)
    def _():
        pltpu.sync_copy(acc_sh, deg_hbm.at[c])


@functools.partial(
    pl.kernel,
    out_type=jax.ShapeDtypeStruct((NC, NPAD // NC // 128, 128), F32),
    scratch_types=[
        pltpu.VMEM((EPAD // EB // NS, EB), I32),
        pltpu.VMEM((NPAD // NC // 128, 128), F32),
        pltpu.VMEM((NPAD // NC // 128,), I32),
        pltpu.VMEM_SHARED((NPAD // NC // 128, 128), F32),
    ],
    **_SC_PARAMS,
)
def _deg(dst_hbm, iden_hbm, deg_hbm, idx_v, hist_v, iden_v, acc_sh):
    _deg_body(dst_hbm, iden_hbm, deg_hbm, idx_v, hist_v, iden_v, acc_sh)


def _mm1_body(deg_ref, x_ref, w1_ref, hs_ref, dis_ref):
    dis = lax.rsqrt(deg_ref[...] + 1.0)
    h = jnp.dot(x_ref[...], w1_ref[...], preferred_element_type=F32)
    hs = h * dis
    hs_ref[0] = hs[:, :128]
    hs_ref[1] = hs[:, 128:]
    dis_ref[...] = dis


def _mm1(deg, xp, w1):
    R = 1024
    g = NPAD // R
    return pl.pallas_call(
        _mm1_body,
        grid=(g,),
        in_specs=[
            pl.BlockSpec((R, 1), lambda i: (i, 0)),
            pl.BlockSpec((R, 128), lambda i: (i, 0)),
            pl.BlockSpec((128, 256), lambda i: (0, 0)),
        ],
        out_specs=[
            pl.BlockSpec((2, R, 128), lambda i: (0, i, 0)),
            pl.BlockSpec((R, 1), lambda i: (i, 0)),
        ],
        out_shape=[
            jax.ShapeDtypeStruct((2, NPAD, 128), F32),
            jax.ShapeDtypeStruct((NPAD, 1), F32),
        ],
    )(deg, xp, w1)


def _agg_pipeline(src_hbm, dst_hbm, tab_hbm, src_v, dst_v, buf0, buf1,
                  acc_sh, g0, g1, s0, s1, base, nb, ch, off):
    def chunk(q, _):
        pltpu.sync_copy(src_hbm.at[pl.ds(base + q * ch, ch)], src_v)
        pltpu.sync_copy(dst_hbm.at[pl.ds(base + q * ch, ch)], dst_v)

        if off is not None:
            def obody(i, _):
                sl = (i // 8, pl.ds((i % 8) * 16, 16))
                src_v[sl] = src_v[sl] + off
                return 0

            lax.fori_loop(0, ch * 8, obody, 0)

        pltpu.async_copy(tab_hbm.at[src_v.at[0]], buf0, g0)
        pltpu.async_copy(tab_hbm.at[src_v.at[1]], buf1, g1)

        def body(j2, _):
            j = 2 * j2
            pltpu.make_async_copy(tab_hbm.at[src_v.at[j]], buf0, g0).wait()
            pltpu.async_copy(buf0, acc_sh.at[dst_v.at[j]], s0, add=True)
            pltpu.make_async_copy(tab_hbm.at[src_v.at[j + 1]], buf1, g1).wait()
            pltpu.async_copy(buf1, acc_sh.at[dst_v.at[j + 1]], s1, add=True)
            pltpu.make_async_copy(buf0, acc_sh.at[dst_v.at[j]], s0).wait()

            @pl.when(j + 2 < ch)
            def _():
                pltpu.async_copy(tab_hbm.at[src_v.at[j + 2]], buf0, g0)

            pltpu.make_async_copy(buf1, acc_sh.at[dst_v.at[j + 1]], s1).wait()

            @pl.when(j + 3 < ch)
            def _():
                pltpu.async_copy(tab_hbm.at[src_v.at[j + 3]], buf1, g1)

            return 0

        lax.fori_loop(0, ch // 2, body, 0)
        return 0

    lax.fori_loop(0, nb // ch, chunk, 0)


def _agg_cols_body(src_hbm, dst_hbm, tab_hbm, out_hbm,
                   src_v, dst_v, buf0, buf1, zbuf_v, acc_sh, g0, g1, s0, s1):
    c = lax.axis_index("c")
    s = lax.axis_index("s")
    nb = EPAD // EB // NS
    rows = NPAD // NS
    _zero_fill(zbuf_v, 256)

    def zcopy(k, _):
        pltpu.sync_copy(zbuf_v, acc_sh.at[pl.ds(s * rows + k * 32, 32)])
        return 0

    lax.fori_loop(0, rows // 32, zcopy, 0)
    plsc.subcore_barrier()
    _agg_pipeline(src_hbm, dst_hbm, tab_hbm, src_v, dst_v, buf0, buf1,
                  acc_sh, g0, g1, s0, s1, s * nb, nb, 32, c * NPAD)
    plsc.subcore_barrier()
    pltpu.sync_copy(acc_sh.at[pl.ds(s * rows, rows)],
                    out_hbm.at[c, pl.ds(s * rows, rows)])


@functools.partial(
    pl.kernel,
    out_type=jax.ShapeDtypeStruct((2, NPAD, 128), F32),
    scratch_types=[
        pltpu.VMEM((32, EB), I32),
        pltpu.VMEM((32, EB), I32),
        pltpu.VMEM((EB, 128), F32),
        pltpu.VMEM((EB, 128), F32),
        pltpu.VMEM((32, 128), F32),
        pltpu.VMEM_SHARED((NPAD, 128), F32),
        pltpu.SemaphoreType.DMA,
        pltpu.SemaphoreType.DMA,
        pltpu.SemaphoreType.DMA,
        pltpu.SemaphoreType.DMA,
    ],
    **_SC_PARAMS,
)
def _agg_cols(src_hbm, dst_hbm, tab_hbm, out_hbm,
              src_v, dst_v, buf0, buf1, zbuf_v, acc_sh, g0, g1, s0, s1):
    _agg_cols_body(src_hbm, dst_hbm, tab_hbm, out_hbm,
                   src_v, dst_v, buf0, buf1, zbuf_v, acc_sh, g0, g1, s0, s1)


def _mm2_body(dis_ref, alo_ref, ahi_ref, hlo_ref, hhi_ref, b1_ref, w2_ref,
              out_ref):
    dis = dis_ref[...]
    b1 = b1_ref[...]
    zlo = jnp.maximum((alo_ref[0] + hlo_ref[0]) * dis + b1[:, :128], 0.0)
    zhi = jnp.maximum((ahi_ref[0] + hhi_ref[0]) * dis + b1[:, 128:], 0.0)
    h2 = (jnp.dot(zlo, w2_ref[...][:128], preferred_element_type=F32)
          + jnp.dot(zhi, w2_ref[...][128:], preferred_element_type=F32))
    out_ref[...] = h2 * dis


def _mm2(dis, agg3, hs3, b1, w2):
    R = 1024
    g = NPAD // R
    return pl.pallas_call(
        _mm2_body,
        grid=(g,),
        in_specs=[
            pl.BlockSpec((R, 1), lambda i: (i, 0)),
            pl.BlockSpec((1, R, 128), lambda i: (0, i, 0)),
            pl.BlockSpec((1, R, 128), lambda i: (1, i, 0)),
            pl.BlockSpec((1, R, 128), lambda i: (0, i, 0)),
            pl.BlockSpec((1, R, 128), lambda i: (1, i, 0)),
            pl.BlockSpec((1, 256), lambda i: (0, 0)),
            pl.BlockSpec((256, 128), lambda i: (0, 0)),
        ],
        out_specs=pl.BlockSpec((R, 128), lambda i: (i, 0)),
        out_shape=jax.ShapeDtypeStruct((NPAD, 128), F32),
    )(dis, agg3, agg3, hs3, hs3, b1, w2)


def _agg_rows_body(src_hbm, dst_hbm, tab_hbm, out_hbm,
                   src_v, dst_v, buf0, buf1, zbuf_v, acc_sh, g0, g1, s0, s1):
    c = lax.axis_index("c")
    s = lax.axis_index("s")
    nb = EPAD // EB // (NC * NS)
    rows = NPAD // NS
    _zero_fill(zbuf_v, 256)

    def zcopy(k, _):
        pltpu.sync_copy(zbuf_v, acc_sh.at[pl.ds(s * rows + k * 32, 32)])
        return 0

    lax.fori_loop(0, rows // 32, zcopy, 0)
    plsc.subcore_barrier()
    _agg_pipeline(src_hbm, dst_hbm, tab_hbm, src_v, dst_v, buf0, buf1,
                  acc_sh, g0, g1, s0, s1, (c * NS + s) * nb, nb, 16, None)
    plsc.subcore_barrier()
    pltpu.sync_copy(acc_sh.at[pl.ds(s * rows, rows)],
                    out_hbm.at[c, pl.ds(s * rows, rows)])


@functools.partial(
    pl.kernel,
    out_type=jax.ShapeDtypeStruct((2, NPAD, 128), F32),
    scratch_types=[
        pltpu.VMEM((16, EB), I32),
        pltpu.VMEM((16, EB), I32),
        pltpu.VMEM((EB, 128), F32),
        pltpu.VMEM((EB, 128), F32),
        pltpu.VMEM((32, 128), F32),
        pltpu.VMEM_SHARED((NPAD, 128), F32),
        pltpu.SemaphoreType.DMA,
        pltpu.SemaphoreType.DMA,
        pltpu.SemaphoreType.DMA,
        pltpu.SemaphoreType.DMA,
    ],
    **_SC_PARAMS,
)
def _agg_rows(src_hbm, dst_hbm, tab_hbm, out_hbm,
              src_v, dst_v, buf0, buf1, zbuf_v, acc_sh, g0, g1, s0, s1):
    _agg_rows_body(src_hbm, dst_hbm, tab_hbm, out_hbm,
                   src_v, dst_v, buf0, buf1, zbuf_v, acc_sh, g0, g1, s0, s1)


def _mm3_body(dis_ref, p0_ref, p1_ref, h2_ref, b2_ref, wt_ref, wb_ref, bl1_ref,
              z_ref, a_ref, bb_ref):
    dis = dis_ref[...]
    z = (p0_ref[0] + p1_ref[0] + h2_ref[...]) * dis + b2_ref[...]
    z_ref[...] = z
    a_ref[...] = jnp.dot(
        z, wt_ref[...], preferred_element_type=F32).astype(jnp.bfloat16)
    bb_ref[...] = (jnp.dot(z, wb_ref[...], preferred_element_type=F32)
                   + bl1_ref[...]).astype(jnp.bfloat16)


def _mm3(dis, p3, h2, b2, wt, wb, bl1):
    R = 1024
    g = NPAD // R
    return pl.pallas_call(
        _mm3_body,
        grid=(g,),
        in_specs=[
            pl.BlockSpec((R, 1), lambda i: (i, 0)),
            pl.BlockSpec((1, R, 128), lambda i: (0, i, 0)),
            pl.BlockSpec((1, R, 128), lambda i: (1, i, 0)),
            pl.BlockSpec((R, 128), lambda i: (i, 0)),
            pl.BlockSpec((1, 128), lambda i: (0, 0)),
            pl.BlockSpec((128, 256), lambda i: (0, 0)),
            pl.BlockSpec((128, 256), lambda i: (0, 0)),
            pl.BlockSpec((1, 256), lambda i: (0, 0)),
        ],
        out_specs=[
            pl.BlockSpec((R, 128), lambda i: (i, 0)),
            pl.BlockSpec((R, 256), lambda i: (i, 0)),
            pl.BlockSpec((R, 256), lambda i: (i, 0)),
        ],
        out_shape=[
            jax.ShapeDtypeStruct((NPAD, 128), F32),
            jax.ShapeDtypeStruct((NPAD, 256), jnp.bfloat16),
            jax.ShapeDtypeStruct((NPAD, 256), jnp.bfloat16),
        ],
    )(dis, p3, p3, h2, b2, wt, wb, bl1)


DB = 128
DNB = 2 * PEPAD // DB // (NC * NS)
BF16 = jnp.bfloat16


def _decode_body(sv_hbm, dv_hbm, a_hbm, b_hbm, wb_hbm, bl2_hbm,
                 out_hbm, sv, dv, bufa0, bufb0, bufa1, bufb1, wbv, bl2v_ref,
                 score_v, sa0, sb0, sa1, sb1):
    c = lax.axis_index("c")
    s = lax.axis_index("s")
    wid = c * NS + s
    pltpu.sync_copy(wb_hbm, wbv)
    pltpu.sync_copy(bl2_hbm, bl2v_ref)
    pltpu.sync_copy(sv_hbm.at[pl.ds(wid * DNB, DNB)], sv)
    pltpu.sync_copy(dv_hbm.at[pl.ds(wid * DNB, DNB)], dv)
    bl2v = bl2v_ref[0]
    lane = lax.iota(I32, 16)
    zero = jnp.zeros((16,), F32)
    wev = [wbv[2 * wc] for wc in range(8)]
    wod = [wbv[2 * wc + 1] for wc in range(8)]

    def compute(j, bufa, bufb):
        def dot1(e):
            acc_e = zero
            acc_o = zero
            for wc in range(8):
                wa = bufa[e, pl.ds(wc * 16, 16)]
                wb = bufb[e, pl.ds(wc * 16, 16)]
                u = jnp.maximum(plsc.bitcast(wa, BF16)
                                + plsc.bitcast(wb, BF16), 0.0)
                m = plsc.bitcast(u, I32)
                ue = plsc.bitcast(lax.shift_left(m, 16), F32)
                uo = plsc.bitcast(m & jnp.int32(-65536), F32)
                acc_e = acc_e + ue * wev[wc]
                acc_o = acc_o + uo * wod[wc]
            return jnp.sum(acc_e + acc_o)

        def ebody(e2, svec):
            e = 4 * e2
            for q in range(4):
                sq = dot1(e + q)
                svec = jnp.where(lane == ((e + q) & 15), sq, svec)

            @pl.when(((e + 3) & 15) == 15)
            def _():
                gidx = lax.shift_left(lax.shift_right_logical(e, 4), 4)
                score_v[j & 15, pl.ds(gidx, 16)] = (
                    1.0 / (1.0 + jnp.exp(-(svec + bl2v))))

            return svec

        lax.fori_loop(0, DB // 4, ebody, zero)

    pltpu.async_copy(a_hbm.at[sv.at[0]], bufa0, sa0)
    pltpu.async_copy(b_hbm.at[dv.at[0]], bufb0, sb0)

    def body(j2, _):
        j = 2 * j2
        pltpu.async_copy(a_hbm.at[sv.at[j + 1]], bufa1, sa1)
        pltpu.async_copy(b_hbm.at[dv.at[j + 1]], bufb1, sb1)
        pltpu.make_async_copy(a_hbm.at[sv.at[j]], bufa0, sa0).wait()
        pltpu.make_async_copy(b_hbm.at[dv.at[j]], bufb0, sb0).wait()
        compute(j, bufa0, bufb0)

        @pl.when(j + 2 < DNB)
        def _():
            pltpu.async_copy(a_hbm.at[sv.at[j + 2]], bufa0, sa0)
            pltpu.async_copy(b_hbm.at[dv.at[j + 2]], bufb0, sb0)

        pltpu.make_async_copy(a_hbm.at[sv.at[j + 1]], bufa1, sa1).wait()
        pltpu.make_async_copy(b_hbm.at[dv.at[j + 1]], bufb1, sb1).wait()
        compute(j + 1, bufa1, bufb1)

        @pl.when((j2 & 7) == 7)
        def _():
            start = pl.multiple_of(wid * DNB + 2 * j2 - 14, 8)
            pltpu.sync_copy(score_v, out_hbm.at[pl.ds(start, 16)])

        return 0

    lax.fori_loop(0, DNB // 2, body, 0)


@functools.partial(
    pl.kernel,
    out_type=jax.ShapeDtypeStruct((2 * PEPAD // DB, DB), F32),
    scratch_types=[
        pltpu.VMEM((DNB, DB), I32),
        pltpu.VMEM((DNB, DB), I32),
        pltpu.VMEM((DB, 128), I32),
        pltpu.VMEM((DB, 128), I32),
        pltpu.VMEM((DB, 128), I32),
        pltpu.VMEM((DB, 128), I32),
        pltpu.VMEM((16, 16), F32),
        pltpu.VMEM((1, 16), F32),
        pltpu.VMEM((16, DB), F32),
        pltpu.SemaphoreType.DMA,
        pltpu.SemaphoreType.DMA,
        pltpu.SemaphoreType.DMA,
        pltpu.SemaphoreType.DMA,
    ],
    **_SC_PARAMS,
)
def _decode(sv_hbm, dv_hbm, a_hbm, b_hbm, wb_hbm, bl2_hbm,
            out_hbm, sv, dv, bufa0, bufb0, bufa1, bufb1, wbv, bl2v_ref,
            score_v, sa0, sb0, sa1, sb1):
    _decode_body(sv_hbm, dv_hbm, a_hbm, b_hbm, wb_hbm, bl2_hbm,
                 out_hbm, sv, dv, bufa0, bufb0, bufa1, bufb1, wbv, bl2v_ref,
                 score_v, sa0, sb0, sa1, sb1)


def kernel(x, edge_index, pos_edge_index, neg_edge_index,
           W1, b1, W2, b2, Wl1, bl1, Wl2, bl2):
    epad = jnp.full((EPAD - E,), N, I32)
    src2 = jnp.concatenate([edge_index[0], epad]).reshape(EPAD // EB, EB)
    dst2 = jnp.concatenate([edge_index[1], epad]).reshape(EPAD // EB, EB)
    xp = jnp.pad(x, ((0, NPAD - N), (0, 0)))

    iden = jnp.arange(NPAD // NC // 128, dtype=I32)
    deg = _deg(dst2, iden).reshape(NPAD, 1)
    hs3, dis = _mm1(deg, xp, W1)
    agg3 = _agg_cols(src2, dst2, hs3.reshape(2 * NPAD, 128))
    h2 = _mm2(dis, agg3, hs3, b1.reshape(1, 256), W2)
    p3 = _agg_rows(src2, dst2, h2)
    z, A, Bb = _mm3(dis, p3, h2, b2.reshape(1, 128), Wl1[:128], Wl1[128:],
                    bl1.reshape(1, 256))

    ppad = jnp.zeros((PEPAD - PE,), I32)
    sv2 = jnp.concatenate([pos_edge_index[0], ppad, neg_edge_index[0], ppad]
                          ).reshape(2 * PEPAD // DB, DB)
    dv2 = jnp.concatenate([pos_edge_index[1], ppad, neg_edge_index[1], ppad]
                          ).reshape(2 * PEPAD // DB, DB)
    apk = lax.bitcast_convert_type(A.reshape(NPAD, 128, 2), I32)
    bpk = lax.bitcast_convert_type(Bb.reshape(NPAD, 128, 2), I32)
    wbc = Wl2[:, 0].reshape(8, 16, 2).transpose(0, 2, 1).reshape(16, 16)
    bl2v = jnp.broadcast_to(bl2, (1, 16))

    scores = _decode(sv2, dv2, apk, bpk, wbc, bl2v).reshape(-1)
    return (z[:N], scores[:PE], scores[PEPAD:PEPAD + PE])
